# Initial kernel scaffold; baseline (speedup 1.0000x reference)
import jax
import jax.numpy as jnp
from jax.experimental import pallas as pl


def kernel(x, table):
    raise NotImplementedError("write your pallas kernel here")


# speedup vs baseline: 7.7468x; 7.7468x over previous
import functools

import jax
import jax.numpy as jnp
from jax import lax
from jax.experimental import pallas as pl
from jax.experimental.pallas import tpu as pltpu
from jax.experimental.pallas import tpu_sc as plsc

EMBED = 128
BATCH = 4096
SEQ = 200

NUM_CORES = 2
NUM_SUBCORES = 16
NUM_WORKERS = NUM_CORES * NUM_SUBCORES
BPW = BATCH // NUM_WORKERS
CHUNK = BPW * SEQ
LANES = 16

_mesh = plsc.VectorSubcoreMesh(core_axis_name="c", subcore_axis_name="s")


@functools.partial(
    pl.kernel,
    mesh=_mesh,
    out_type=(
        jax.ShapeDtypeStruct((SEQ * BATCH, EMBED), jnp.float32),
        jax.ShapeDtypeStruct((BATCH * SEQ,), jnp.float32),
    ),
    scratch_types=[
        pltpu.VMEM((CHUNK,), jnp.int32),
        pltpu.VMEM((CHUNK,), jnp.float32),
        pltpu.VMEM((BPW,), jnp.int32),
        pltpu.VMEM((BPW,), jnp.int32),
        pltpu.VMEM((BPW, EMBED), jnp.float32),
        pltpu.VMEM((BPW, EMBED), jnp.float32),
        pltpu.SemaphoreType.DMA,
        pltpu.SemaphoreType.DMA,
        pltpu.SemaphoreType.DMA,
        pltpu.SemaphoreType.DMA,
    ],
    compiler_params=pltpu.CompilerParams(needs_layout_passes=False),
)
def _lookup(x_hbm, table_hbm, out_hbm, mask_hbm,
            x_v, mask_v, idx0, idx1, rows0, rows1,
            gsem0, gsem1, wsem, msem):
    wid = lax.axis_index("s") * NUM_CORES + lax.axis_index("c")
    base = wid * CHUNK
    b0 = wid * BPW

    pltpu.sync_copy(x_hbm.at[pl.ds(base, CHUNK)], x_v)

    def mask_body(t, c):
        j0 = t * LANES
        v = x_v[pl.ds(j0, LANES)]
        mask_v[pl.ds(j0, LANES)] = jnp.where(
            v != 0, jnp.float32(1.0), jnp.float32(0.0))
        return c

    lax.fori_loop(0, CHUNK // LANES, mask_body, 0)
    mask_cp = pltpu.async_copy(mask_v, mask_hbm.at[pl.ds(base, CHUNK)], msem)

    def build_idx(s, idx_ref):
        for g in range(BPW // LANES):
            bvec = lax.iota(jnp.int32, LANES) + (g * LANES)
            vals = plsc.load_gather(x_v, [bvec * SEQ + s])
            idx_ref[pl.ds(g * LANES, LANES)] = vals

    def body(i, c):
        s0 = 2 * i
        s1 = s0 + 1
        build_idx(s0, idx0)
        g0 = pltpu.async_copy(table_hbm.at[idx0], rows0, gsem0)
        build_idx(s1, idx1)
        g1 = pltpu.async_copy(table_hbm.at[idx1], rows1, gsem1)
        g0.wait()
        w0 = pltpu.async_copy(
            rows0, out_hbm.at[pl.ds(s0 * BATCH + b0, BPW), :], wsem)
        g1.wait()
        w1 = pltpu.async_copy(
            rows1, out_hbm.at[pl.ds(s1 * BATCH + b0, BPW), :], wsem)
        w0.wait()
        w1.wait()
        return c

    lax.fori_loop(0, SEQ // 2, body, 0)
    mask_cp.wait()


def kernel(x, table):
    out_rows, mask_flat = _lookup(x.reshape(-1), table)
    return (out_rows.reshape(SEQ, BATCH, EMBED), mask_flat.reshape(BATCH, SEQ))


# speedup vs baseline: 9.0246x; 1.1649x over previous
import functools

import jax
import jax.numpy as jnp
from jax import lax
from jax.experimental import pallas as pl
from jax.experimental.pallas import tpu as pltpu
from jax.experimental.pallas import tpu_sc as plsc

EMBED = 128
BATCH = 4096
SEQ = 200

NUM_CORES = 2
NUM_SUBCORES = 16
NUM_WORKERS = NUM_CORES * NUM_SUBCORES
BPW = BATCH // NUM_WORKERS
CHUNK = BPW * SEQ
LANES = 16

N_SLOTS = 4
ROUNDS = SEQ // N_SLOTS
MASK_VECS = CHUNK // LANES
MASK_VECS_PER_ROUND = MASK_VECS // ROUNDS

_mesh = plsc.VectorSubcoreMesh(core_axis_name="c", subcore_axis_name="s")


@functools.partial(
    pl.kernel,
    mesh=_mesh,
    out_type=(
        jax.ShapeDtypeStruct((SEQ * BATCH, EMBED), jnp.float32),
        jax.ShapeDtypeStruct((BATCH * SEQ,), jnp.float32),
    ),
    scratch_types=[
        pltpu.VMEM((CHUNK,), jnp.int32),
        pltpu.VMEM((CHUNK,), jnp.float32),
        [pltpu.VMEM((BPW,), jnp.int32)] * N_SLOTS,
        [pltpu.VMEM((BPW, EMBED), jnp.float32)] * N_SLOTS,
        [pltpu.SemaphoreType.DMA] * N_SLOTS,
        [pltpu.SemaphoreType.DMA] * N_SLOTS,
        pltpu.SemaphoreType.DMA,
    ],
    compiler_params=pltpu.CompilerParams(needs_layout_passes=False),
)
def _lookup(x_hbm, table_hbm, out_hbm, mask_hbm,
            x_v, mask_v, idx, rows, gsem, wsem, msem):
    wid = lax.axis_index("s") * NUM_CORES + lax.axis_index("c")
    base = wid * CHUNK
    b0 = wid * BPW

    pltpu.sync_copy(x_hbm.at[pl.ds(base, CHUNK)], x_v)

    def build_idx(s, idx_ref):
        for g in range(BPW // LANES):
            bvec = lax.iota(jnp.int32, LANES) + (g * LANES)
            vals = plsc.load_gather(x_v, [bvec * SEQ + s])
            idx_ref[pl.ds(g * LANES, LANES)] = vals

    def out_slice(s):
        return out_hbm.at[pl.ds(s * BATCH + b0, BPW), :]

    def body(i, c):
        for j in range(N_SLOTS):
            s = i * N_SLOTS + j

            @pl.when(i > 0)
            def _():
                pltpu.make_async_copy(rows[j], out_slice(s), wsem[j]).wait()

            build_idx(s, idx[j])
            pltpu.async_copy(table_hbm.at[idx[j]], rows[j], gsem[j])

        m0 = i * MASK_VECS_PER_ROUND
        for t in range(MASK_VECS_PER_ROUND):
            j0 = (m0 + t) * LANES
            v = x_v[pl.ds(j0, LANES)]
            mask_v[pl.ds(j0, LANES)] = jnp.where(
                v != 0, jnp.float32(1.0), jnp.float32(0.0))

        for j in range(N_SLOTS):
            s = i * N_SLOTS + j
            pltpu.make_async_copy(table_hbm.at[idx[j]], rows[j], gsem[j]).wait()
            pltpu.async_copy(rows[j], out_slice(s), wsem[j])
        return c

    lax.fori_loop(0, ROUNDS, body, 0)

    mask_cp = pltpu.async_copy(mask_v, mask_hbm.at[pl.ds(base, CHUNK)], msem)
    for j in range(N_SLOTS):
        s = SEQ - N_SLOTS + j
        pltpu.make_async_copy(rows[j], out_slice(s), wsem[j]).wait()
    mask_cp.wait()


def kernel(x, table):
    out_rows, mask_flat = _lookup(x.reshape(-1), table)
    return (out_rows.reshape(SEQ, BATCH, EMBED), mask_flat.reshape(BATCH, SEQ))
